# baseline (speedup 1.0000x reference)
import jax
import jax.numpy as jnp
from jax.experimental import pallas as pl

NUM_Q = 8
INPUT_DIM = 512
CODE_DIM = 256
KSIZE = 2048
T = 8192

HT = 256
NH = 2
BT = NH * HT


def _esq_kernel(emb_ref, esq_ref):
    e = emb_ref[...]
    esq_ref[...] = jnp.sum(e * e, axis=-1)


def _rvq_kernel(x_ref, win_ref, wout_ref, emb2_ref, esq_ref,
                out_ref, codes_ref):
    iota = jax.lax.broadcasted_iota(jnp.int32, (HT, KSIZE), 1)
    res = [x_ref[h * HT:(h + 1) * HT] for h in range(NH)]
    idx_rows = [[] for _ in range(NH)]

    def stage_front(h, q):
        xp = jax.lax.dot_general(
            res[h], win_ref[q], (((1,), (1,)), ((), ())),
            preferred_element_type=jnp.float32)
        x_sq = jnp.sum(xp * xp, axis=-1, keepdims=True)
        cross2 = jax.lax.dot_general(
            xp, emb2_ref[q], (((1,), (1,)), ((), ())),
            preferred_element_type=jnp.float32)
        dist = (x_sq - cross2) + esq_ref[q][None, :]
        di = jax.lax.bitcast_convert_type(dist, jnp.int32)
        base = jax.lax.bitcast_convert_type(x_sq, jnp.int32)
        key = (di - base) * KSIZE + iota
        minkey = jnp.min(key, axis=-1, keepdims=True)
        return jnp.bitwise_and(minkey, KSIZE - 1)

    def stage_back(h, q, idx):
        onehot = (iota == idx).astype(jnp.float32)
        quant2 = jax.lax.dot_general(
            onehot, emb2_ref[q], (((1,), (0,)), ((), ())),
            preferred_element_type=jnp.float32)
        dec = jax.lax.dot_general(
            quant2, wout_ref[q], (((1,), (1,)), ((), ())),
            preferred_element_type=jnp.float32)
        res[h] = res[h] - dec

    for q in range(NUM_Q):
        idxs = [stage_front(h, q) for h in range(NH)]
        for h in range(NH):
            idx_rows[h].append(idxs[h])
            stage_back(h, q, idxs[h])

    for h in range(NH):
        out_ref[h * HT:(h + 1) * HT] = x_ref[h * HT:(h + 1) * HT] - res[h]
    codes_ref[...] = jnp.concatenate(
        [jnp.concatenate([idx_rows[h][q].reshape(1, HT)
                          for h in range(NH)], axis=1)
         for q in range(NUM_Q)], axis=0)


def kernel(x_td, Win_qcd, Wout_qdc, emb_qkc):
    emb2 = (emb_qkc * 2.0).astype(jnp.bfloat16)
    wout_half = (Wout_qdc * 0.5).astype(jnp.bfloat16)
    win = Win_qcd.astype(jnp.bfloat16)
    esq_qk = pl.pallas_call(
        _esq_kernel,
        out_shape=jax.ShapeDtypeStruct((NUM_Q, KSIZE), jnp.float32),
    )(emb_qkc)

    grid = (T // BT,)
    out_td, codes_qt = pl.pallas_call(
        _rvq_kernel,
        grid=grid,
        in_specs=[
            pl.BlockSpec((BT, INPUT_DIM), lambda i: (i, 0)),
            pl.BlockSpec((NUM_Q, CODE_DIM, INPUT_DIM), lambda i: (0, 0, 0)),
            pl.BlockSpec((NUM_Q, INPUT_DIM, CODE_DIM), lambda i: (0, 0, 0)),
            pl.BlockSpec((NUM_Q, KSIZE, CODE_DIM), lambda i: (0, 0, 0)),
            pl.BlockSpec((NUM_Q, KSIZE), lambda i: (0, 0)),
        ],
        out_specs=(
            pl.BlockSpec((BT, INPUT_DIM), lambda i: (i, 0)),
            pl.BlockSpec((NUM_Q, BT), lambda i: (0, i)),
        ),
        out_shape=(
            jax.ShapeDtypeStruct((T, INPUT_DIM), jnp.float32),
            jax.ShapeDtypeStruct((NUM_Q, T), jnp.int32),
        ),
    )(x_td, win, wout_half, emb2, esq_qk)
    return out_td, codes_qt


# speedup vs baseline: 1.0275x; 1.0275x over previous
import jax
import jax.numpy as jnp
from jax.experimental import pallas as pl

NUM_Q = 8
INPUT_DIM = 512
CODE_DIM = 256
KSIZE = 2048
T = 8192

HT = 256
NH = 2
BT = NH * HT


def _esq_kernel(emb_ref, esq_ref):
    e = emb_ref[...]
    esq_ref[...] = jnp.sum(e * e, axis=-1)


def _rvq_kernel(x_ref, win_ref, wout_ref, emb2_ref, esq_ref,
                out_ref, codes_ref):
    iota = jax.lax.broadcasted_iota(jnp.int32, (HT, KSIZE), 1)
    res = [x_ref[h * HT:(h + 1) * HT] for h in range(NH)]
    idx_rows = [[] for _ in range(NH)]

    def stage_front(h, q):
        xp = jax.lax.dot_general(
            res[h].astype(jnp.bfloat16), win_ref[q], (((1,), (1,)), ((), ())),
            preferred_element_type=jnp.float32)
        x_sq = jnp.sum(xp * xp, axis=-1, keepdims=True)
        cross2 = jax.lax.dot_general(
            xp.astype(jnp.bfloat16), emb2_ref[q], (((1,), (1,)), ((), ())),
            preferred_element_type=jnp.float32)
        dist = (x_sq - cross2) + esq_ref[q][None, :]
        di = jax.lax.bitcast_convert_type(dist, jnp.int32)
        base = jax.lax.bitcast_convert_type(x_sq, jnp.int32)
        key = (di - base) * KSIZE + iota
        minkey = jnp.min(key, axis=-1, keepdims=True)
        return jnp.bitwise_and(minkey, KSIZE - 1)

    def stage_back(h, q, idx):
        onehot = (iota == idx).astype(jnp.bfloat16)
        quant2 = jax.lax.dot_general(
            onehot, emb2_ref[q], (((1,), (0,)), ((), ())),
            preferred_element_type=jnp.float32)
        dec = jax.lax.dot_general(
            quant2.astype(jnp.bfloat16), wout_ref[q], (((1,), (1,)), ((), ())),
            preferred_element_type=jnp.float32)
        res[h] = res[h] - dec

    for q in range(NUM_Q):
        idxs = [stage_front(h, q) for h in range(NH)]
        for h in range(NH):
            idx_rows[h].append(idxs[h])
            stage_back(h, q, idxs[h])

    for h in range(NH):
        out_ref[h * HT:(h + 1) * HT] = x_ref[h * HT:(h + 1) * HT] - res[h]
    codes_ref[...] = jnp.concatenate(
        [jnp.concatenate([idx_rows[h][q].reshape(1, HT)
                          for h in range(NH)], axis=1)
         for q in range(NUM_Q)], axis=0)


def kernel(x_td, Win_qcd, Wout_qdc, emb_qkc):
    emb2 = (emb_qkc * 2.0).astype(jnp.bfloat16)
    wout_half = (Wout_qdc * 0.5).astype(jnp.bfloat16)
    win = Win_qcd.astype(jnp.bfloat16)
    esq_qk = pl.pallas_call(
        _esq_kernel,
        out_shape=jax.ShapeDtypeStruct((NUM_Q, KSIZE), jnp.float32),
    )(emb_qkc)

    grid = (T // BT,)
    out_td, codes_qt = pl.pallas_call(
        _rvq_kernel,
        grid=grid,
        in_specs=[
            pl.BlockSpec((BT, INPUT_DIM), lambda i: (i, 0)),
            pl.BlockSpec((NUM_Q, CODE_DIM, INPUT_DIM), lambda i: (0, 0, 0)),
            pl.BlockSpec((NUM_Q, INPUT_DIM, CODE_DIM), lambda i: (0, 0, 0)),
            pl.BlockSpec((NUM_Q, KSIZE, CODE_DIM), lambda i: (0, 0, 0)),
            pl.BlockSpec((NUM_Q, KSIZE), lambda i: (0, 0)),
        ],
        out_specs=(
            pl.BlockSpec((BT, INPUT_DIM), lambda i: (i, 0)),
            pl.BlockSpec((NUM_Q, BT), lambda i: (0, i)),
        ),
        out_shape=(
            jax.ShapeDtypeStruct((T, INPUT_DIM), jnp.float32),
            jax.ShapeDtypeStruct((NUM_Q, T), jnp.int32),
        ),
    )(x_td, win, wout_half, emb2, esq_qk)
    return out_td, codes_qt


# speedup vs baseline: 1.0910x; 1.0618x over previous
import jax
import jax.numpy as jnp
from jax.experimental import pallas as pl

NUM_Q = 8
INPUT_DIM = 512
CODE_DIM = 256
KSIZE = 2048
T = 8192

HT = 512
NH = 2
BT = NH * HT


def _esq_kernel(emb_ref, esq_ref):
    e = emb_ref[...]
    esq_ref[...] = jnp.sum(e * e, axis=-1)


def _rvq_kernel(x_ref, win_ref, wout_ref, emb2_ref, esq_ref,
                out_ref, codes_ref):
    iota = jax.lax.broadcasted_iota(jnp.int32, (HT, KSIZE), 1)
    res = [x_ref[h * HT:(h + 1) * HT] for h in range(NH)]
    idx_rows = [[] for _ in range(NH)]

    def stage_front(h, q):
        xp = jax.lax.dot_general(
            res[h].astype(jnp.bfloat16), win_ref[q], (((1,), (1,)), ((), ())),
            preferred_element_type=jnp.float32)
        x_sq = jnp.sum(xp * xp, axis=-1, keepdims=True)
        cross2 = jax.lax.dot_general(
            xp.astype(jnp.bfloat16), emb2_ref[q], (((1,), (1,)), ((), ())),
            preferred_element_type=jnp.float32)
        dist = (x_sq - cross2) + esq_ref[q][None, :]
        di = jax.lax.bitcast_convert_type(dist, jnp.int32)
        base = jax.lax.bitcast_convert_type(x_sq, jnp.int32)
        key = (di - base) * KSIZE + iota
        minkey = jnp.min(key, axis=-1, keepdims=True)
        return jnp.bitwise_and(minkey, KSIZE - 1)

    def stage_back(h, q, idx):
        onehot = (iota == idx).astype(jnp.bfloat16)
        quant2 = jax.lax.dot_general(
            onehot, emb2_ref[q], (((1,), (0,)), ((), ())),
            preferred_element_type=jnp.float32)
        dec = jax.lax.dot_general(
            quant2.astype(jnp.bfloat16), wout_ref[q], (((1,), (1,)), ((), ())),
            preferred_element_type=jnp.float32)
        res[h] = res[h] - dec

    for q in range(NUM_Q):
        idxs = [stage_front(h, q) for h in range(NH)]
        for h in range(NH):
            idx_rows[h].append(idxs[h])
            stage_back(h, q, idxs[h])

    for h in range(NH):
        out_ref[h * HT:(h + 1) * HT] = x_ref[h * HT:(h + 1) * HT] - res[h]
    codes_ref[...] = jnp.concatenate(
        [jnp.concatenate([idx_rows[h][q].reshape(1, HT)
                          for h in range(NH)], axis=1)
         for q in range(NUM_Q)], axis=0)


def kernel(x_td, Win_qcd, Wout_qdc, emb_qkc):
    emb2 = (emb_qkc * 2.0).astype(jnp.bfloat16)
    wout_half = (Wout_qdc * 0.5).astype(jnp.bfloat16)
    win = Win_qcd.astype(jnp.bfloat16)
    esq_qk = pl.pallas_call(
        _esq_kernel,
        out_shape=jax.ShapeDtypeStruct((NUM_Q, KSIZE), jnp.float32),
    )(emb_qkc)

    grid = (T // BT,)
    out_td, codes_qt = pl.pallas_call(
        _rvq_kernel,
        grid=grid,
        in_specs=[
            pl.BlockSpec((BT, INPUT_DIM), lambda i: (i, 0)),
            pl.BlockSpec((NUM_Q, CODE_DIM, INPUT_DIM), lambda i: (0, 0, 0)),
            pl.BlockSpec((NUM_Q, INPUT_DIM, CODE_DIM), lambda i: (0, 0, 0)),
            pl.BlockSpec((NUM_Q, KSIZE, CODE_DIM), lambda i: (0, 0, 0)),
            pl.BlockSpec((NUM_Q, KSIZE), lambda i: (0, 0)),
        ],
        out_specs=(
            pl.BlockSpec((BT, INPUT_DIM), lambda i: (i, 0)),
            pl.BlockSpec((NUM_Q, BT), lambda i: (0, i)),
        ),
        out_shape=(
            jax.ShapeDtypeStruct((T, INPUT_DIM), jnp.float32),
            jax.ShapeDtypeStruct((NUM_Q, T), jnp.int32),
        ),
    )(x_td, win, wout_half, emb2, esq_qk)
    return out_td, codes_qt
